# baseline (speedup 1.0000x reference)
import jax
import jax.numpy as jnp
from jax import lax
from jax.experimental import pallas as pl
from jax.experimental.pallas import tpu as pltpu
from jax.experimental.pallas import tpu_sc as plsc

N = 10000
E = 320000
C = 128
NC, NS, LN = 2, 16, 16
NW = NC * NS
B = 80
EPAD = 327680
EPW = EPAD // NW
NB = EPW // B
NPAD = 10240
PADN = 10200
NPT = NPAD // NS
FCH = 40
NFC = NPT // FCH


def _node_tables(y):
    blk = 2000

    def body(y_ref, l_ref, p_ref, a_ref):
        x = y_ref[...]
        m = jnp.max(x, axis=1, keepdims=True)
        xm = x - m
        ex = jnp.exp(xm)
        sex = jnp.sum(ex, axis=1, keepdims=True)
        lsm = xm - jnp.log(sex)
        p = ex / sex
        l_ref[...] = lsm
        p_ref[...] = p
        a_ref[...] = jnp.sum(p * lsm, axis=1, keepdims=True)

    def imap(i):
        return (i, jnp.asarray(0, i.dtype) if hasattr(i, "dtype") else 0)

    return pl.pallas_call(
        body,
        grid=(N // blk,),
        in_specs=[pl.BlockSpec((blk, C), imap)],
        out_specs=[
            pl.BlockSpec((blk, C), imap),
            pl.BlockSpec((blk, C), imap),
            pl.BlockSpec((blk, 1), imap),
        ],
        out_shape=[
            jax.ShapeDtypeStruct((N, C), jnp.float32),
            jax.ShapeDtypeStruct((N, C), jnp.float32),
            jax.ShapeDtypeStruct((N, 1), jnp.float32),
        ],
    )(y)


def _sc_body(p_hbm, l_hbm, a_hbm, src_hbm, dst_hbm, w_hbm,
             cross_out, asum_out, colsum_out,
             L0, L1, sr0, sr1, dr0, dr1, wr0, wr1, ones,
             gch, pch, abuf, cntbuf, csbuf, zbuf, stage,
             g_sp, cnt_sh, colsum_sh,
             semG0, semG1, semS):
    def _i32(x):
        if getattr(x, "dtype", None) == jnp.int32:
            return x
        return jnp.asarray(x, jnp.int32)

    c = _i32(lax.axis_index("c"))
    s = _i32(lax.axis_index("s"))
    wid = c * NS + s

    fzero = jnp.zeros((LN,), jnp.float32)
    fone = jnp.full((LN,), 1.0, jnp.float32)

    e0 = wid * EPW
    row0 = s * NPT

    for i in range(B // LN):
        ones[pl.ds(i * LN, LN)] = fone

    def zrow_step(r, carry):
        r = _i32(r)
        for j in range(C // LN):
            gch[r, pl.ds(j * LN, LN)] = fzero
        return carry

    lax.fori_loop(jnp.int32(0), jnp.int32(FCH), zrow_step, jnp.int32(0))

    def zch_step(ch, carry):
        ch = _i32(ch)
        pltpu.sync_copy(gch, g_sp.at[pl.ds(row0 + ch * FCH, FCH)])
        return carry

    lax.fori_loop(jnp.int32(0), jnp.int32(NFC), zch_step, jnp.int32(0))
    for i in range(NPT // LN):
        zbuf[pl.ds(i * LN, LN)] = fzero
    pltpu.sync_copy(zbuf, cnt_sh.at[pl.ds(row0, NPT)])
    pltpu.sync_copy(zbuf, colsum_sh.at[pl.ds(row0, NPT)])
    plsc.subcore_barrier()

    srs = (sr0, sr1)
    drs = (dr0, dr1)
    wrs = (wr0, wr1)
    Lb = (L0, L1)
    semG = (semG0, semG1)

    def load_idx(batch, slot):
        off = e0 + _i32(batch) * B
        pltpu.sync_copy(src_hbm.at[pl.ds(off, B)], srs[slot].at[_i32(0)])
        pltpu.sync_copy(dst_hbm.at[pl.ds(off, B)], drs[slot].at[_i32(0)])
        pltpu.sync_copy(w_hbm.at[pl.ds(off, B)], wrs[slot].at[_i32(0)])

    def fire_gather(slot):
        pltpu.async_copy(l_hbm.at[drs[slot].at[_i32(0)]], Lb[slot],
                         semG[slot])

    def wait_gather(slot):
        pltpu.make_async_copy(l_hbm.at[drs[slot].at[_i32(0)]], Lb[slot],
                              semG[slot]).wait()

    def fire_scatters(slot):
        z = _i32(0)
        pltpu.async_copy(Lb[slot], g_sp.at[srs[slot].at[z]], semS, add=True)
        pltpu.async_copy(wrs[slot].at[z], colsum_sh.at[drs[slot].at[z]],
                         semS, add=True)
        pltpu.async_copy(ones, cnt_sh.at[srs[slot].at[z]], semS, add=True)

    def drain_scatters(slot):
        z = _i32(0)
        pltpu.make_async_copy(Lb[slot], g_sp.at[srs[slot].at[z]], semS).wait()
        pltpu.make_async_copy(wrs[slot].at[z], colsum_sh.at[drs[slot].at[z]],
                              semS).wait()
        pltpu.make_async_copy(ones, cnt_sh.at[srs[slot].at[z]], semS).wait()

    load_idx(0, 0)
    load_idx(1, 1)
    fire_gather(0)
    fire_gather(1)

    def edge_step(g2, carry):
        g = g2 * 2
        for b in range(2):
            k = g + b
            wait_gather(b)
            fire_scatters(b)
            drain_scatters(b)

            @pl.when(k + 2 <= NB - 1)
            def _():
                load_idx(k + 2, b)
                fire_gather(b)
        return carry

    lax.fori_loop(jnp.int32(0), jnp.int32(NB // 2), edge_step, jnp.int32(0))
    plsc.subcore_barrier()

    def chunk_step(ch, accs):
        r0 = row0 + _i32(ch) * FCH
        pltpu.sync_copy(g_sp.at[pl.ds(r0, FCH)], gch)
        pltpu.sync_copy(p_hbm.at[pl.ds(r0, FCH)], pch)

        def frow(r, a8):
            r = _i32(r)
            return tuple(
                a8[j] + gch[r, pl.ds(j * LN, LN)] * pch[r, pl.ds(j * LN, LN)]
                for j in range(C // LN))

        return lax.fori_loop(jnp.int32(0), jnp.int32(FCH), frow, accs)

    accs0 = tuple(fzero for _ in range(C // LN))
    accs = lax.fori_loop(jnp.int32(0), jnp.int32(NFC), chunk_step, accs0)
    crossv = accs[0]
    for j in range(1, C // LN):
        crossv = crossv + accs[j]

    pltpu.sync_copy(cnt_sh.at[pl.ds(row0, NPT)], cntbuf)
    pltpu.sync_copy(a_hbm.at[pl.ds(row0, NPT)], abuf)

    def arow(i, acc):
        i = _i32(i)
        return acc + cntbuf[pl.ds(i * LN, LN)] * abuf[pl.ds(i * LN, LN)]

    aacc = lax.fori_loop(jnp.int32(0), jnp.int32(NPT // LN), arow, fzero)

    stage[...] = crossv
    pltpu.sync_copy(stage, cross_out.at[wid])
    stage[...] = aacc
    pltpu.sync_copy(stage, asum_out.at[wid])

    pltpu.sync_copy(colsum_sh.at[pl.ds(row0, NPT)], csbuf)
    pltpu.sync_copy(csbuf, colsum_out.at[c, pl.ds(row0, NPT)])


def _edge_terms(p_pad, l_pad, a_pad, srcf, dstf, wf):
    mesh = plsc.VectorSubcoreMesh(core_axis_name="c", subcore_axis_name="s")
    f32 = jnp.float32
    i32 = jnp.int32
    return pl.kernel(
        _sc_body,
        out_type=[
            jax.ShapeDtypeStruct((NW, LN), f32),
            jax.ShapeDtypeStruct((NW, LN), f32),
            jax.ShapeDtypeStruct((NC, NPAD), f32),
        ],
        mesh=mesh,
        compiler_params=pltpu.CompilerParams(needs_layout_passes=False),
        scratch_types=[
            pltpu.VMEM((B, C), f32),
            pltpu.VMEM((B, C), f32),
            pltpu.VMEM((1, B), i32),
            pltpu.VMEM((1, B), i32),
            pltpu.VMEM((1, B), i32),
            pltpu.VMEM((1, B), i32),
            pltpu.VMEM((1, B), f32),
            pltpu.VMEM((1, B), f32),
            pltpu.VMEM((B,), f32),
            pltpu.VMEM((FCH, C), f32),
            pltpu.VMEM((FCH, C), f32),
            pltpu.VMEM((NPT,), f32),
            pltpu.VMEM((NPT,), f32),
            pltpu.VMEM((NPT,), f32),
            pltpu.VMEM((NPT,), f32),
            pltpu.VMEM((LN,), f32),
            pltpu.VMEM_SHARED((NPAD, C), f32),
            pltpu.VMEM_SHARED((NPAD,), f32),
            pltpu.VMEM_SHARED((NPAD,), f32),
            pltpu.SemaphoreType.DMA,
            pltpu.SemaphoreType.DMA,
            pltpu.SemaphoreType.DMA,
        ],
    )(p_pad, l_pad, a_pad, srcf, dstf, wf)


def kernel(y_1, edge_index, edge_weight):
    y = y_1.astype(jnp.float32)
    src = edge_index[0].astype(jnp.int32)
    dst = edge_index[1].astype(jnp.int32)
    w = edge_weight.astype(jnp.float32)
    npad = EPAD - E
    srcf = jnp.concatenate([src, jnp.full((npad,), PADN, jnp.int32)])
    dstf = jnp.concatenate([dst, jnp.full((npad,), PADN, jnp.int32)])
    wf = jnp.concatenate([w, jnp.zeros((npad,), jnp.float32)])

    lsm, p, a2 = _node_tables(y)
    zrow = jnp.zeros((NPAD - N, C), jnp.float32)
    p_pad = jnp.concatenate([p, zrow])
    l_pad = jnp.concatenate([lsm, zrow])
    a_pad = jnp.concatenate([a2.reshape(N), jnp.zeros((NPAD - N,),
                                                      jnp.float32)])

    cross_p, asum_p, colsum_p = _edge_terms(p_pad, l_pad, a_pad,
                                            srcf, dstf, wf)

    cross = jnp.sum(cross_p)
    asum = jnp.sum(asum_p)
    colsum = colsum_p[0] + colsum_p[1]
    s_count = jnp.sum(jnp.where(colsum > 0, 1.0, 0.0).astype(jnp.float32))
    kl_scalar = (asum - cross) / jnp.float32(E)
    ncr = s_count * kl_scalar / jnp.float32(N)
    return ncr.astype(jnp.float32)


# speedup vs baseline: 2.6209x; 2.6209x over previous
import jax
import jax.numpy as jnp
from jax import lax
from jax.experimental import pallas as pl
from jax.experimental.pallas import tpu as pltpu
from jax.experimental.pallas import tpu_sc as plsc

N = 10000
E = 320000
C = 128
NC, NS, LN = 2, 16, 16
NW = NC * NS
B = 80
EPW = E // NW
NB = EPW // B
EPAD = 327680
CROWS = EPAD // 128
R_PS = CROWS // NS
CG = 8
NPAD = 10240
NPT = NPAD // NS


def _node_tables(y):
    blk = 2000

    def body(y_ref, l_ref, p_ref, a_ref):
        x = y_ref[...]
        m = jnp.max(x, axis=1, keepdims=True)
        xm = x - m
        ex = jnp.exp(xm)
        sex = jnp.sum(ex, axis=1, keepdims=True)
        lsm = xm - jnp.log(sex)
        p = ex / sex
        l_ref[...] = lsm
        p_ref[...] = p
        a_ref[...] = jnp.sum(p * lsm, axis=1, keepdims=True)

    def imap(i):
        return (i, jnp.asarray(0, i.dtype) if hasattr(i, "dtype") else 0)

    return pl.pallas_call(
        body,
        grid=(N // blk,),
        in_specs=[pl.BlockSpec((blk, C), imap)],
        out_specs=[
            pl.BlockSpec((blk, C), imap),
            pl.BlockSpec((blk, C), imap),
            pl.BlockSpec((blk, 1), imap),
        ],
        out_shape=[
            jax.ShapeDtypeStruct((N, C), jnp.float32),
            jax.ShapeDtypeStruct((N, C), jnp.float32),
            jax.ShapeDtypeStruct((N, 1), jnp.float32),
        ],
    )(y)


def _sc_body(p_hbm, l_hbm, a_hbm, src_hbm, dst_hbm, dst2_hbm, w2_hbm,
             cross_out, asum_out, scnt_out,
             a_tab, P0, P1, L0, L1, sbuf, dbuf, wchunk, dchunk,
             zbuf, cbuf, stage, colsum_sh,
             semP0, semP1, semL0, semL1, semC):
    def _i32(x):
        if getattr(x, "dtype", None) == jnp.int32:
            return x
        return jnp.asarray(x, jnp.int32)

    c = _i32(lax.axis_index("c"))
    s = _i32(lax.axis_index("s"))
    wid = c * NS + s

    fzero = jnp.zeros((LN,), jnp.float32)
    fone = jnp.full((LN,), 1.0, jnp.float32)

    e0 = wid * EPW
    d_atab = pltpu.async_copy(a_hbm, a_tab, semP0)
    d_sbuf = pltpu.async_copy(src_hbm.at[pl.ds(e0, EPW)], sbuf, semL0)
    d_dbuf = pltpu.async_copy(dst_hbm.at[pl.ds(e0, EPW)], dbuf, semP1)
    d_dch = pltpu.async_copy(dst2_hbm.at[pl.ds(s * R_PS, R_PS)], dchunk, semL1)
    d_wch = pltpu.async_copy(w2_hbm.at[pl.ds(s * R_PS, R_PS)], wchunk, semL1)

    for i in range(NPT // LN):
        zbuf[pl.ds(i * LN, LN)] = fzero
    pltpu.sync_copy(zbuf, colsum_sh.at[pl.ds(s * NPT, NPT)])
    plsc.subcore_barrier()

    d_sbuf.wait()
    d_dbuf.wait()

    def fire0(batch, slot):
        off = jnp.int32(batch * B)
        pltpu.async_copy(p_hbm.at[sbuf.at[pl.ds(off, B)]], P0 if slot == 0
                         else P1, semP0 if slot == 0 else semP1)
        pltpu.async_copy(l_hbm.at[dbuf.at[pl.ds(off, B)]], L0 if slot == 0
                         else L1, semL0 if slot == 0 else semL1)

    fire0(0, 0)
    fire0(1, 1)

    d_dch.wait()
    d_wch.wait()

    def colsum_group(g, carry):
        g = _i32(g)
        for j in range(CG):
            k = g * CG + j
            pltpu.async_copy(wchunk.at[k], colsum_sh.at[dchunk.at[k]],
                             semC, add=True)
        z = _i32(0)
        for j in range(CG):
            pltpu.make_async_copy(
                wchunk.at[z], colsum_sh.at[dchunk.at[z]], semC).wait()
        return carry

    lax.fori_loop(jnp.int32(0), jnp.int32(R_PS // CG), colsum_group,
                  jnp.int32(0))
    plsc.subcore_barrier()

    Pb = (P0, P1)
    Lb = (L0, L1)
    semP = (semP0, semP1)
    semL = (semL0, semL1)

    def fire(batch, slot):
        off = _i32(batch) * B
        pltpu.async_copy(p_hbm.at[sbuf.at[pl.ds(off, B)]], Pb[slot], semP[slot])
        pltpu.async_copy(l_hbm.at[dbuf.at[pl.ds(off, B)]], Lb[slot], semL[slot])

    def wait(slot):
        z = _i32(0)
        pltpu.make_async_copy(
            p_hbm.at[sbuf.at[pl.ds(z, B)]], Pb[slot], semP[slot]).wait()
        pltpu.make_async_copy(
            l_hbm.at[dbuf.at[pl.ds(z, B)]], Lb[slot], semL[slot]).wait()

    def batch_compute(batch, slot, carry):
        off = _i32(batch) * B
        accs, aacc = carry
        P_, L_ = Pb[slot], Lb[slot]
        for i in range(B // LN):
            idxv = sbuf[pl.ds(off + i * LN, LN)]
            aacc = aacc + plsc.load_gather(a_tab, [idxv])

        def row_step(e, a8):
            e = _i32(e)
            return tuple(
                a8[j] + P_[e, pl.ds(j * LN, LN)] * L_[e, pl.ds(j * LN, LN)]
                for j in range(C // LN))

        accs = lax.fori_loop(jnp.int32(0), jnp.int32(B), row_step, accs)
        return accs, aacc

    d_atab.wait()

    accs0 = tuple(fzero for _ in range(C // LN))
    carry0 = (accs0, fzero)

    def ring_step(g2, carry):
        g = g2 * 2
        wait(0)
        fire(g + 2, 0)
        carry = batch_compute(g, 0, carry)
        wait(1)

        @pl.when(g2 <= (NB - 5) // 2)
        def _():
            fire(g + 3, 1)

        carry = batch_compute(g + 1, 1, carry)
        return carry

    carry = lax.fori_loop(jnp.int32(0), jnp.int32((NB - 1) // 2), ring_step,
                          carry0)
    wait(0)
    accs, aacc = batch_compute(NB - 1, 0, carry)

    crossv = accs[0]
    for j in range(1, C // LN):
        crossv = crossv + accs[j]

    stage[...] = crossv
    pltpu.sync_copy(stage, cross_out.at[wid])
    stage[...] = aacc
    pltpu.sync_copy(stage, asum_out.at[wid])

    pltpu.sync_copy(colsum_sh.at[pl.ds(s * NPT, NPT)], cbuf)

    def cnt_step(i, cnt):
        v = cbuf[pl.ds(i * LN, LN)]
        return cnt + jnp.where(v > 0.0, fone, fzero)

    cnt = lax.fori_loop(jnp.int32(0), jnp.int32(NPT // LN), cnt_step, fzero)
    stage[...] = cnt
    pltpu.sync_copy(stage, scnt_out.at[wid])


def _edge_terms(p, lsm, a, src, dst, dst2, w2):
    mesh = plsc.VectorSubcoreMesh(core_axis_name="c", subcore_axis_name="s")
    f32 = jnp.float32
    i32 = jnp.int32
    return pl.kernel(
        _sc_body,
        out_type=[
            jax.ShapeDtypeStruct((NW, LN), f32),
            jax.ShapeDtypeStruct((NW, LN), f32),
            jax.ShapeDtypeStruct((NW, LN), f32),
        ],
        mesh=mesh,
        compiler_params=pltpu.CompilerParams(needs_layout_passes=False),
        scratch_types=[
            pltpu.VMEM((N,), f32),
            pltpu.VMEM((B, C), f32),
            pltpu.VMEM((B, C), f32),
            pltpu.VMEM((B, C), f32),
            pltpu.VMEM((B, C), f32),
            pltpu.VMEM((EPW,), i32),
            pltpu.VMEM((EPW,), i32),
            pltpu.VMEM((R_PS, 128), f32),
            pltpu.VMEM((R_PS, 128), i32),
            pltpu.VMEM((NPT,), f32),
            pltpu.VMEM((NPT,), f32),
            pltpu.VMEM((LN,), f32),
            pltpu.VMEM_SHARED((NPAD,), f32),
            pltpu.SemaphoreType.DMA,
            pltpu.SemaphoreType.DMA,
            pltpu.SemaphoreType.DMA,
            pltpu.SemaphoreType.DMA,
            pltpu.SemaphoreType.DMA,
        ],
    )(p, lsm, a, src, dst, dst2, w2)


def kernel(y_1, edge_index, edge_weight):
    y = y_1.astype(jnp.float32)
    src = edge_index[0].astype(jnp.int32)
    dst = edge_index[1].astype(jnp.int32)
    w = edge_weight.astype(jnp.float32)
    npad = EPAD - E
    dst2 = jnp.concatenate([dst, jnp.zeros((npad,), jnp.int32)]).reshape(
        CROWS, 128)
    w2 = jnp.concatenate([w, jnp.zeros((npad,), jnp.float32)]).reshape(
        CROWS, 128)

    lsm, p, a2 = _node_tables(y)
    a = a2.reshape(N)

    cross_p, asum_p, scnt_p = _edge_terms(p, lsm, a, src, dst, dst2, w2)

    cross = jnp.sum(cross_p)
    asum = jnp.sum(asum_p)
    s_count = jnp.sum(scnt_p[:NS])
    kl_scalar = (asum - cross) / jnp.float32(E)
    ncr = s_count * kl_scalar / jnp.float32(N)
    return ncr.astype(jnp.float32)


# speedup vs baseline: 2.7696x; 1.0567x over previous
import jax
import jax.numpy as jnp
from jax import lax
from jax.experimental import pallas as pl
from jax.experimental.pallas import tpu as pltpu
from jax.experimental.pallas import tpu_sc as plsc

N = 10000
E = 320000
C = 128
NC, NS, LN = 2, 16, 16
NW = NC * NS
B = 80
EPW = E // NW
NB = EPW // B
EPAD = 327680
CROWS = EPAD // 128
R_PS = CROWS // NS
CG = 8
NPAD = 10240
NPT = NPAD // NS


def _node_tables(y):
    blk = 2000

    def body(y_ref, l_ref, p_ref, a_ref):
        x = y_ref[...]
        m = jnp.max(x, axis=1, keepdims=True)
        xm = x - m
        ex = jnp.exp(xm)
        sex = jnp.sum(ex, axis=1, keepdims=True)
        lsm = xm - jnp.log(sex)
        p = ex / sex
        l_ref[...] = lsm
        p_ref[...] = p
        a_ref[...] = jnp.sum(p * lsm, axis=1, keepdims=True)

    def imap(i):
        return (i, jnp.asarray(0, i.dtype) if hasattr(i, "dtype") else 0)

    return pl.pallas_call(
        body,
        grid=(N // blk,),
        in_specs=[pl.BlockSpec((blk, C), imap)],
        out_specs=[
            pl.BlockSpec((blk, C), imap),
            pl.BlockSpec((blk, C), imap),
            pl.BlockSpec((blk, 1), imap),
        ],
        out_shape=[
            jax.ShapeDtypeStruct((N, C), jnp.float32),
            jax.ShapeDtypeStruct((N, C), jnp.float32),
            jax.ShapeDtypeStruct((N, 1), jnp.float32),
        ],
    )(y)


def _sc_body(p_hbm, l_hbm, a_hbm, src_hbm, dst_hbm, dst2_hbm, w2_hbm,
             cross_out, asum_out, scnt_out,
             a_tab, P0, P1, L0, L1, sbuf, dbuf, wchunk, dchunk,
             zbuf, cbuf, stage, colsum_sh,
             semP0, semP1, semL0, semL1, semC):
    def _i32(x):
        if getattr(x, "dtype", None) == jnp.int32:
            return x
        return jnp.asarray(x, jnp.int32)

    c = _i32(lax.axis_index("c"))
    s = _i32(lax.axis_index("s"))
    wid = c * NS + s

    fzero = jnp.zeros((LN,), jnp.float32)
    fone = jnp.full((LN,), 1.0, jnp.float32)

    e0 = wid * EPW
    d_atab = pltpu.async_copy(a_hbm, a_tab, semP0)
    d_sbuf = pltpu.async_copy(src_hbm.at[pl.ds(e0, EPW)], sbuf, semL0)
    d_dbuf = pltpu.async_copy(dst_hbm.at[pl.ds(e0, EPW)], dbuf, semP1)
    d_dch = pltpu.async_copy(dst2_hbm.at[pl.ds(s * R_PS, R_PS)], dchunk, semL1)
    d_wch = pltpu.async_copy(w2_hbm.at[pl.ds(s * R_PS, R_PS)], wchunk, semL1)

    for i in range(NPT // LN):
        zbuf[pl.ds(i * LN, LN)] = fzero
    pltpu.sync_copy(zbuf, colsum_sh.at[pl.ds(s * NPT, NPT)])
    plsc.subcore_barrier()

    d_sbuf.wait()
    d_dbuf.wait()

    def fire0(batch, slot):
        off = jnp.int32(batch * B)
        pltpu.async_copy(p_hbm.at[sbuf.at[pl.ds(off, B)]], P0 if slot == 0
                         else P1, semP0 if slot == 0 else semP1)
        pltpu.async_copy(l_hbm.at[dbuf.at[pl.ds(off, B)]], L0 if slot == 0
                         else L1, semL0 if slot == 0 else semL1)

    fire0(0, 0)
    fire0(1, 1)

    d_dch.wait()
    d_wch.wait()

    Pb = (P0, P1)
    Lb = (L0, L1)
    semP = (semP0, semP1)
    semL = (semL0, semL1)

    def fire(batch, slot):
        off = _i32(batch) * B
        pltpu.async_copy(p_hbm.at[sbuf.at[pl.ds(off, B)]], Pb[slot], semP[slot])
        pltpu.async_copy(l_hbm.at[dbuf.at[pl.ds(off, B)]], Lb[slot], semL[slot])

    def wait(slot):
        z = _i32(0)
        pltpu.make_async_copy(
            p_hbm.at[sbuf.at[pl.ds(z, B)]], Pb[slot], semP[slot]).wait()
        pltpu.make_async_copy(
            l_hbm.at[dbuf.at[pl.ds(z, B)]], Lb[slot], semL[slot]).wait()

    def batch_compute(batch, slot, carry):
        off = _i32(batch) * B
        accs, aacc = carry
        P_, L_ = Pb[slot], Lb[slot]
        for i in range(B // LN):
            idxv = sbuf[pl.ds(off + i * LN, LN)]
            aacc = aacc + plsc.load_gather(a_tab, [idxv])

        def row_step(e, a8):
            e = _i32(e)
            return tuple(
                a8[j] + P_[e, pl.ds(j * LN, LN)] * L_[e, pl.ds(j * LN, LN)]
                for j in range(C // LN))

        accs = lax.fori_loop(jnp.int32(0), jnp.int32(B), row_step, accs)
        return accs, aacc

    d_atab.wait()

    accs0 = tuple(fzero for _ in range(C // LN))
    carry0 = (accs0, fzero)

    CPI = 4
    NCI = R_PS // CPI

    def ring_step(g2, carry):
        g = g2 * 2
        wait(0)
        fire(g + 2, 0)

        @pl.when(g2 <= NCI - 1)
        def _():
            for j in range(CPI):
                k = g2 * CPI + j
                pltpu.async_copy(wchunk.at[k], colsum_sh.at[dchunk.at[k]],
                                 semC, add=True)

        carry = batch_compute(g, 0, carry)
        wait(1)

        @pl.when(g2 <= (NB - 5) // 2)
        def _():
            fire(g + 3, 1)

        carry = batch_compute(g + 1, 1, carry)

        @pl.when(g2 <= NCI - 1)
        def _():
            z = _i32(0)
            for j in range(CPI):
                pltpu.make_async_copy(
                    wchunk.at[z], colsum_sh.at[dchunk.at[z]], semC).wait()

        return carry

    carry = lax.fori_loop(jnp.int32(0), jnp.int32((NB - 1) // 2), ring_step,
                          carry0)
    wait(0)
    accs, aacc = batch_compute(NB - 1, 0, carry)
    plsc.subcore_barrier()

    crossv = accs[0]
    for j in range(1, C // LN):
        crossv = crossv + accs[j]

    stage[...] = crossv
    pltpu.sync_copy(stage, cross_out.at[wid])
    stage[...] = aacc
    pltpu.sync_copy(stage, asum_out.at[wid])

    pltpu.sync_copy(colsum_sh.at[pl.ds(s * NPT, NPT)], cbuf)

    def cnt_step(i, cnt):
        v = cbuf[pl.ds(i * LN, LN)]
        return cnt + jnp.where(v > 0.0, fone, fzero)

    cnt = lax.fori_loop(jnp.int32(0), jnp.int32(NPT // LN), cnt_step, fzero)
    stage[...] = cnt
    pltpu.sync_copy(stage, scnt_out.at[wid])


def _edge_terms(p, lsm, a, src, dst, dst2, w2):
    mesh = plsc.VectorSubcoreMesh(core_axis_name="c", subcore_axis_name="s")
    f32 = jnp.float32
    i32 = jnp.int32
    return pl.kernel(
        _sc_body,
        out_type=[
            jax.ShapeDtypeStruct((NW, LN), f32),
            jax.ShapeDtypeStruct((NW, LN), f32),
            jax.ShapeDtypeStruct((NW, LN), f32),
        ],
        mesh=mesh,
        compiler_params=pltpu.CompilerParams(needs_layout_passes=False),
        scratch_types=[
            pltpu.VMEM((N,), f32),
            pltpu.VMEM((B, C), f32),
            pltpu.VMEM((B, C), f32),
            pltpu.VMEM((B, C), f32),
            pltpu.VMEM((B, C), f32),
            pltpu.VMEM((EPW,), i32),
            pltpu.VMEM((EPW,), i32),
            pltpu.VMEM((R_PS, 128), f32),
            pltpu.VMEM((R_PS, 128), i32),
            pltpu.VMEM((NPT,), f32),
            pltpu.VMEM((NPT,), f32),
            pltpu.VMEM((LN,), f32),
            pltpu.VMEM_SHARED((NPAD,), f32),
            pltpu.SemaphoreType.DMA,
            pltpu.SemaphoreType.DMA,
            pltpu.SemaphoreType.DMA,
            pltpu.SemaphoreType.DMA,
            pltpu.SemaphoreType.DMA,
        ],
    )(p, lsm, a, src, dst, dst2, w2)


def kernel(y_1, edge_index, edge_weight):
    y = y_1.astype(jnp.float32)
    src = edge_index[0].astype(jnp.int32)
    dst = edge_index[1].astype(jnp.int32)
    w = edge_weight.astype(jnp.float32)
    npad = EPAD - E
    dst2 = jnp.concatenate([dst, jnp.zeros((npad,), jnp.int32)]).reshape(
        CROWS, 128)
    w2 = jnp.concatenate([w, jnp.zeros((npad,), jnp.float32)]).reshape(
        CROWS, 128)

    lsm, p, a2 = _node_tables(y)
    a = a2.reshape(N)

    cross_p, asum_p, scnt_p = _edge_terms(p, lsm, a, src, dst, dst2, w2)

    cross = jnp.sum(cross_p)
    asum = jnp.sum(asum_p)
    s_count = jnp.sum(scnt_p[:NS])
    kl_scalar = (asum - cross) / jnp.float32(E)
    ncr = s_count * kl_scalar / jnp.float32(N)
    return ncr.astype(jnp.float32)


# speedup vs baseline: 2.7714x; 1.0006x over previous
import jax
import jax.numpy as jnp
from jax import lax
from jax.experimental import pallas as pl
from jax.experimental.pallas import tpu as pltpu
from jax.experimental.pallas import tpu_sc as plsc

N = 10000
E = 320000
C = 128
NC, NS, LN = 2, 16, 16
NW = NC * NS
B = 80
EPW = E // NW
NB = EPW // B
EPAD = 327680
CROWS = EPAD // 128
R_PS = CROWS // NS
CG = 8
NPAD = 10240
NPT = NPAD // NS


def _node_tables(y):
    blk = 2000

    def body(y_ref, l_ref, p_ref, a_ref):
        x = y_ref[...]
        m = jnp.max(x, axis=1, keepdims=True)
        xm = x - m
        ex = jnp.exp(xm)
        sex = jnp.sum(ex, axis=1, keepdims=True)
        lsm = xm - jnp.log(sex)
        p = ex / sex
        l_ref[...] = lsm
        p_ref[...] = p
        a_ref[...] = jnp.sum(p * lsm, axis=1, keepdims=True)

    def imap(i):
        return (i, jnp.asarray(0, i.dtype) if hasattr(i, "dtype") else 0)

    return pl.pallas_call(
        body,
        grid=(N // blk,),
        in_specs=[pl.BlockSpec((blk, C), imap)],
        out_specs=[
            pl.BlockSpec((blk, C), imap),
            pl.BlockSpec((blk, C), imap),
            pl.BlockSpec((blk, 1), imap),
        ],
        out_shape=[
            jax.ShapeDtypeStruct((N, C), jnp.float32),
            jax.ShapeDtypeStruct((N, C), jnp.float32),
            jax.ShapeDtypeStruct((N, 1), jnp.float32),
        ],
    )(y)


def _sc_body(p_hbm, l_hbm, a_hbm, src_hbm, dst_hbm, dst2_hbm, w2_hbm,
             cross_out, asum_out, scnt_out,
             a_tab, P0, P1, L0, L1, sbuf, dbuf, wchunk, dchunk,
             zbuf, cbuf, stage, colsum_sh,
             semP0, semP1, semL0, semL1, semC):
    def _i32(x):
        if getattr(x, "dtype", None) == jnp.int32:
            return x
        return jnp.asarray(x, jnp.int32)

    c = _i32(lax.axis_index("c"))
    s = _i32(lax.axis_index("s"))
    wid = c * NS + s

    fzero = jnp.zeros((LN,), jnp.float32)
    fone = jnp.full((LN,), 1.0, jnp.float32)

    e0 = wid * EPW
    d_atab = pltpu.async_copy(a_hbm, a_tab, semP0)
    d_sbuf = pltpu.async_copy(src_hbm.at[pl.ds(e0, EPW)], sbuf, semL0)
    d_dbuf = pltpu.async_copy(dst_hbm.at[pl.ds(e0, EPW)], dbuf, semP1)
    d_dch = pltpu.async_copy(dst2_hbm.at[pl.ds(s * R_PS, R_PS)], dchunk, semL1)
    d_wch = pltpu.async_copy(w2_hbm.at[pl.ds(s * R_PS, R_PS)], wchunk, semL1)

    for i in range(NPT // LN):
        zbuf[pl.ds(i * LN, LN)] = fzero
    pltpu.sync_copy(zbuf, colsum_sh.at[pl.ds(s * NPT, NPT)])
    plsc.subcore_barrier()

    d_sbuf.wait()
    d_dbuf.wait()

    def fire0(batch, slot):
        off = jnp.int32(batch * B)
        pltpu.async_copy(p_hbm.at[sbuf.at[pl.ds(off, B)]], P0 if slot == 0
                         else P1, semP0 if slot == 0 else semP1)
        pltpu.async_copy(l_hbm.at[dbuf.at[pl.ds(off, B)]], L0 if slot == 0
                         else L1, semL0 if slot == 0 else semL1)

    fire0(0, 0)
    fire0(1, 1)

    d_dch.wait()
    d_wch.wait()

    Pb = (P0, P1)
    Lb = (L0, L1)
    semP = (semP0, semP1)
    semL = (semL0, semL1)

    def fire(batch, slot):
        off = _i32(batch) * B
        pltpu.async_copy(p_hbm.at[sbuf.at[pl.ds(off, B)]], Pb[slot], semP[slot])
        pltpu.async_copy(l_hbm.at[dbuf.at[pl.ds(off, B)]], Lb[slot], semL[slot])

    def wait(slot):
        z = _i32(0)
        pltpu.make_async_copy(
            p_hbm.at[sbuf.at[pl.ds(z, B)]], Pb[slot], semP[slot]).wait()
        pltpu.make_async_copy(
            l_hbm.at[dbuf.at[pl.ds(z, B)]], Lb[slot], semL[slot]).wait()

    def batch_compute(batch, slot, carry):
        off = _i32(batch) * B
        accs, aacc = carry
        P_, L_ = Pb[slot], Lb[slot]
        for i in range(B // LN):
            idxv = sbuf[pl.ds(off + i * LN, LN)]
            aacc = aacc + plsc.load_gather(a_tab, [idxv])

        def row_step(e2, a8):
            for r in range(2):
                e = _i32(e2) * 2 + r
                a8 = tuple(
                    a8[j] + P_[e, pl.ds(j * LN, LN)] * L_[e, pl.ds(j * LN, LN)]
                    for j in range(C // LN))
            return a8

        accs = lax.fori_loop(jnp.int32(0), jnp.int32(B // 2), row_step, accs)
        return accs, aacc

    d_atab.wait()

    accs0 = tuple(fzero for _ in range(C // LN))
    carry0 = (accs0, fzero)

    CPI = 4
    NCI = R_PS // CPI

    def ring_step(g2, carry):
        g = g2 * 2
        wait(0)
        fire(g + 2, 0)

        @pl.when(g2 <= NCI - 1)
        def _():
            for j in range(CPI):
                k = g2 * CPI + j
                pltpu.async_copy(wchunk.at[k], colsum_sh.at[dchunk.at[k]],
                                 semC, add=True)

        carry = batch_compute(g, 0, carry)
        wait(1)

        @pl.when(g2 <= (NB - 5) // 2)
        def _():
            fire(g + 3, 1)

        carry = batch_compute(g + 1, 1, carry)

        @pl.when(g2 <= NCI - 1)
        def _():
            z = _i32(0)
            for j in range(CPI):
                pltpu.make_async_copy(
                    wchunk.at[z], colsum_sh.at[dchunk.at[z]], semC).wait()

        return carry

    carry = lax.fori_loop(jnp.int32(0), jnp.int32((NB - 1) // 2), ring_step,
                          carry0)
    wait(0)
    accs, aacc = batch_compute(NB - 1, 0, carry)
    plsc.subcore_barrier()

    crossv = accs[0]
    for j in range(1, C // LN):
        crossv = crossv + accs[j]

    stage[...] = crossv
    pltpu.sync_copy(stage, cross_out.at[wid])
    stage[...] = aacc
    pltpu.sync_copy(stage, asum_out.at[wid])

    pltpu.sync_copy(colsum_sh.at[pl.ds(s * NPT, NPT)], cbuf)

    def cnt_step(i, cnt):
        v = cbuf[pl.ds(i * LN, LN)]
        return cnt + jnp.where(v > 0.0, fone, fzero)

    cnt = lax.fori_loop(jnp.int32(0), jnp.int32(NPT // LN), cnt_step, fzero)
    stage[...] = cnt
    pltpu.sync_copy(stage, scnt_out.at[wid])


def _edge_terms(p, lsm, a, src, dst, dst2, w2):
    mesh = plsc.VectorSubcoreMesh(core_axis_name="c", subcore_axis_name="s")
    f32 = jnp.float32
    i32 = jnp.int32
    return pl.kernel(
        _sc_body,
        out_type=[
            jax.ShapeDtypeStruct((NW, LN), f32),
            jax.ShapeDtypeStruct((NW, LN), f32),
            jax.ShapeDtypeStruct((NW, LN), f32),
        ],
        mesh=mesh,
        compiler_params=pltpu.CompilerParams(needs_layout_passes=False),
        scratch_types=[
            pltpu.VMEM((N,), f32),
            pltpu.VMEM((B, C), f32),
            pltpu.VMEM((B, C), f32),
            pltpu.VMEM((B, C), f32),
            pltpu.VMEM((B, C), f32),
            pltpu.VMEM((EPW,), i32),
            pltpu.VMEM((EPW,), i32),
            pltpu.VMEM((R_PS, 128), f32),
            pltpu.VMEM((R_PS, 128), i32),
            pltpu.VMEM((NPT,), f32),
            pltpu.VMEM((NPT,), f32),
            pltpu.VMEM((LN,), f32),
            pltpu.VMEM_SHARED((NPAD,), f32),
            pltpu.SemaphoreType.DMA,
            pltpu.SemaphoreType.DMA,
            pltpu.SemaphoreType.DMA,
            pltpu.SemaphoreType.DMA,
            pltpu.SemaphoreType.DMA,
        ],
    )(p, lsm, a, src, dst, dst2, w2)


def kernel(y_1, edge_index, edge_weight):
    y = y_1.astype(jnp.float32)
    src = edge_index[0].astype(jnp.int32)
    dst = edge_index[1].astype(jnp.int32)
    w = edge_weight.astype(jnp.float32)
    npad = EPAD - E
    dst2 = jnp.concatenate([dst, jnp.zeros((npad,), jnp.int32)]).reshape(
        CROWS, 128)
    w2 = jnp.concatenate([w, jnp.zeros((npad,), jnp.float32)]).reshape(
        CROWS, 128)

    lsm, p, a2 = _node_tables(y)
    a = a2.reshape(N)

    cross_p, asum_p, scnt_p = _edge_terms(p, lsm, a, src, dst, dst2, w2)

    cross = jnp.sum(cross_p)
    asum = jnp.sum(asum_p)
    s_count = jnp.sum(scnt_p[:NS])
    kl_scalar = (asum - cross) / jnp.float32(E)
    ncr = s_count * kl_scalar / jnp.float32(N)
    return ncr.astype(jnp.float32)
